# baseline (speedup 1.0000x reference)
import functools

import jax
import jax.numpy as jnp
from jax import lax
from jax.experimental import pallas as pl
from jax.experimental.pallas import tpu as pltpu
from jax.experimental.pallas import tpu_sc as plsc

D_MODEL = 128
SEQ_LEN = 200
HALF = 100
LANES = 16
NVEC = D_MODEL // LANES
EPS = 1e-5

_GATHER_DNUMS = lax.GatherDimensionNumbers(
    offset_dims=(), collapsed_slice_dims=(0,), start_index_map=(0,))


def _permute(v, idx):
    return lax.gather(
        v, idx[:, None], dimension_numbers=_GATHER_DNUMS, slice_sizes=(1,),
        mode=lax.GatherScatterMode.PROMISE_IN_BOUNDS)


def _lane_sum(v):
    idx = lax.iota(jnp.int32, LANES)
    for d in (8, 4, 2, 1):
        v = v + _permute(v, idx ^ d)
    return v


def _rsqrt(y):
    i = lax.bitcast_convert_type(y, jnp.int32)
    i = jnp.int32(0x5F3759DF) - (i >> 1)
    r = lax.bitcast_convert_type(i, jnp.float32)
    half = y * 0.5
    for _ in range(2):
        r = r * (1.5 - half * r * r)
    return r


def _make_sc_kernel(n_seq):
    info = plsc.get_sparse_core_info()
    nw = info.num_cores * info.num_subcores
    seq_per_w = n_seq // nw
    assert seq_per_w % 2 == 0
    mesh = plsc.VectorSubcoreMesh(core_axis_name="c", subcore_axis_name="s")

    @functools.partial(
        pl.kernel,
        out_type=jax.ShapeDtypeStruct((n_seq * SEQ_LEN, D_MODEL), jnp.float32),
        mesh=mesh,
        scratch_types=[
            pltpu.VMEM((seq_per_w, 2, HALF), jnp.int32),
            pltpu.VMEM((3, SEQ_LEN, D_MODEL), jnp.float32),
            pltpu.VMEM((SEQ_LEN, D_MODEL), jnp.float32),
            pltpu.VMEM((D_MODEL,), jnp.float32),
            pltpu.VMEM((D_MODEL,), jnp.float32),
            pltpu.SemaphoreType.DMA((3,)),
            pltpu.SemaphoreType.DMA((3,)),
        ],
    )
    def sc_kernel(x_hbm, table_hbm, pe_hbm, gamma_hbm, beta_hbm, out_hbm,
                  idx_v, bufs, pe_v, g_v, b_v, gsem, osem):
        wid = lax.axis_index("s") * info.num_cores + lax.axis_index("c")
        base_seq = wid * seq_per_w

        pltpu.sync_copy(x_hbm.at[wid], idx_v)
        pltpu.sync_copy(pe_hbm, pe_v)
        pltpu.sync_copy(gamma_hbm, g_v)
        pltpu.sync_copy(beta_hbm, b_v)

        def start_gather(s, b):
            for h in range(2):
                pltpu.async_copy(
                    table_hbm.at[idx_v.at[s, h]],
                    bufs.at[b, pl.ds(h * HALF, HALF)], gsem.at[b])

        def wait_gather(b):
            pltpu.make_async_copy(
                out_hbm.at[pl.ds(0, SEQ_LEN)], bufs.at[b], gsem.at[b]).wait()

        def start_out(s, b):
            pltpu.async_copy(
                bufs.at[b],
                out_hbm.at[pl.ds((base_seq + s) * SEQ_LEN, SEQ_LEN)],
                osem.at[b])

        def wait_out(b):
            pltpu.make_async_copy(
                bufs.at[b], out_hbm.at[pl.ds(0, SEQ_LEN)], osem.at[b]).wait()

        def compute_seq(b, carry):
            @plsc.parallel_loop(0, SEQ_LEN, unroll=1, carry=carry)
            def row_body(r, c):
                gs = [g_v[pl.ds(j * LANES, LANES)] for j in range(NVEC)]
                bs = [b_v[pl.ds(j * LANES, LANES)] for j in range(NVEC)]
                v = [bufs[b, r, pl.ds(j * LANES, LANES)]
                     + pe_v[r, pl.ds(j * LANES, LANES)]
                     for j in range(NVEC)]
                s1 = (v[0] + v[1]) + (v[2] + v[3])
                s2 = (v[4] + v[5]) + (v[6] + v[7])
                tot = _lane_sum(s1 + s2)
                q1 = (v[0] * v[0] + v[1] * v[1]) + (v[2] * v[2] + v[3] * v[3])
                q2 = (v[4] * v[4] + v[5] * v[5]) + (v[6] * v[6] + v[7] * v[7])
                qtot = _lane_sum(q1 + q2)
                mean = tot * (1.0 / D_MODEL)
                var = qtot * (1.0 / D_MODEL) - mean * mean
                rstd = _rsqrt(var + EPS)
                for j in range(NVEC):
                    bufs[b, r, pl.ds(j * LANES, LANES)] = (
                        (v[j] - mean) * rstd * gs[j] + bs[j])
                return c

            return row_body

        gb = jnp.int32(0)

        start_gather(0, 0)
        n_main = (seq_per_w - 2) // 3

        def triple_body(i, c):
            for b in range(3):
                s = 3 * i + b
                nb = (b + 1) % 3
                if b == 2:
                    wait_out(nb)
                else:
                    @pl.when(i > 0)
                    def _():
                        wait_out(nb)
                start_gather(s + 1, nb)
                wait_gather(b)
                c = compute_seq(b, c)
                start_out(s, b)
            return c

        c = lax.fori_loop(0, n_main, triple_body, gb)
        for s in range(3 * n_main, seq_per_w):
            b = s % 3
            if s + 1 < seq_per_w:
                nb = (s + 1) % 3
                wait_out(nb)
                start_gather(s + 1, nb)
            wait_gather(b)
            c = compute_seq(b, c)
            start_out(s, b)
        for b in range(3):
            wait_out(b)

    return sc_kernel


def kernel(x, table, pe, gamma, beta):
    n_seq, seq_len = x.shape
    assert seq_len == SEQ_LEN
    info = plsc.get_sparse_core_info()
    nw = info.num_cores * info.num_subcores
    xc = x.astype(jnp.int32).reshape(nw, n_seq // nw, 2, HALF)
    pe2 = pe[0, :SEQ_LEN, :]
    out = _make_sc_kernel(n_seq)(xc, table, pe2, gamma, beta)
    return out.reshape(n_seq, seq_len, D_MODEL)


# speedup vs baseline: 1.1228x; 1.1228x over previous
import functools

import jax
import jax.numpy as jnp
from jax import lax
from jax.experimental import pallas as pl
from jax.experimental.pallas import tpu as pltpu
from jax.experimental.pallas import tpu_sc as plsc

D_MODEL = 128
SEQ_LEN = 200
HALF = 100
LANES = 16
NVEC = D_MODEL // LANES
EPS = 1e-5

_GATHER_DNUMS = lax.GatherDimensionNumbers(
    offset_dims=(), collapsed_slice_dims=(0,), start_index_map=(0,))


def _permute(v, idx):
    return lax.gather(
        v, idx[:, None], dimension_numbers=_GATHER_DNUMS, slice_sizes=(1,),
        mode=lax.GatherScatterMode.PROMISE_IN_BOUNDS)


def _lane_sum(v):
    idx = lax.iota(jnp.int32, LANES)
    for d in (8, 4, 2, 1):
        v = v + _permute(v, idx ^ d)
    return v


def _rsqrt(y):
    i = lax.bitcast_convert_type(y, jnp.int32)
    i = jnp.int32(0x5F3759DF) - (i >> 1)
    r = lax.bitcast_convert_type(i, jnp.float32)
    half = y * 0.5
    r = r * (1.5 - half * r * r)
    return r


def _make_sc_kernel(n_seq):
    info = plsc.get_sparse_core_info()
    nw = info.num_cores * info.num_subcores
    seq_per_w = n_seq // nw
    assert seq_per_w % 2 == 0
    mesh = plsc.VectorSubcoreMesh(core_axis_name="c", subcore_axis_name="s")

    @functools.partial(
        pl.kernel,
        out_type=jax.ShapeDtypeStruct((n_seq * SEQ_LEN, D_MODEL), jnp.float32),
        mesh=mesh,
        scratch_types=[
            pltpu.VMEM((seq_per_w, 2, HALF), jnp.int32),
            pltpu.VMEM((3, SEQ_LEN, D_MODEL), jnp.float32),
            pltpu.VMEM((SEQ_LEN, D_MODEL), jnp.float32),
            pltpu.VMEM((D_MODEL,), jnp.float32),
            pltpu.VMEM((D_MODEL,), jnp.float32),
            pltpu.SemaphoreType.DMA((3,)),
            pltpu.SemaphoreType.DMA((3,)),
        ],
    )
    def sc_kernel(x_hbm, table_hbm, pe_hbm, gamma_hbm, beta_hbm, out_hbm,
                  idx_v, bufs, pe_v, g_v, b_v, gsem, osem):
        wid = lax.axis_index("s") * info.num_cores + lax.axis_index("c")
        base_seq = wid * seq_per_w

        pltpu.sync_copy(x_hbm.at[wid], idx_v)
        pltpu.sync_copy(pe_hbm, pe_v)
        pltpu.sync_copy(gamma_hbm, g_v)
        pltpu.sync_copy(beta_hbm, b_v)

        def start_gather(s, b):
            for h in range(2):
                pltpu.async_copy(
                    table_hbm.at[idx_v.at[s, h]],
                    bufs.at[b, pl.ds(h * HALF, HALF)], gsem.at[b])

        def wait_gather(b):
            pltpu.make_async_copy(
                out_hbm.at[pl.ds(0, SEQ_LEN)], bufs.at[b], gsem.at[b]).wait()

        def start_out(s, b):
            pltpu.async_copy(
                bufs.at[b],
                out_hbm.at[pl.ds((base_seq + s) * SEQ_LEN, SEQ_LEN)],
                osem.at[b])

        def wait_out(b):
            pltpu.make_async_copy(
                bufs.at[b], out_hbm.at[pl.ds(0, SEQ_LEN)], osem.at[b]).wait()

        def compute_seq(b, carry):
            @plsc.parallel_loop(0, SEQ_LEN, unroll=1, carry=carry)
            def row_body(r, c):
                gs = c[:NVEC]
                bs = c[NVEC:]
                v = [bufs[b, r, pl.ds(j * LANES, LANES)]
                     + pe_v[r, pl.ds(j * LANES, LANES)]
                     for j in range(NVEC)]
                s1 = (v[0] + v[1]) + (v[2] + v[3])
                s2 = (v[4] + v[5]) + (v[6] + v[7])
                tot = _lane_sum(s1 + s2)
                q1 = (v[0] * v[0] + v[1] * v[1]) + (v[2] * v[2] + v[3] * v[3])
                q2 = (v[4] * v[4] + v[5] * v[5]) + (v[6] * v[6] + v[7] * v[7])
                qtot = _lane_sum(q1 + q2)
                mean = tot * (1.0 / D_MODEL)
                var = qtot * (1.0 / D_MODEL) - mean * mean
                rstd = _rsqrt(var + EPS)
                for j in range(NVEC):
                    bufs[b, r, pl.ds(j * LANES, LANES)] = (
                        (v[j] - mean) * rstd * gs[j] + bs[j])
                return c

            return row_body

        gb = tuple(g_v[pl.ds(j * LANES, LANES)] for j in range(NVEC)) + \
             tuple(b_v[pl.ds(j * LANES, LANES)] for j in range(NVEC))

        start_gather(0, 0)
        n_main = (seq_per_w - 2) // 3

        def triple_body(i, c):
            for b in range(3):
                s = 3 * i + b
                nb = (b + 1) % 3
                if b == 2:
                    wait_out(nb)
                else:
                    @pl.when(i > 0)
                    def _():
                        wait_out(nb)
                start_gather(s + 1, nb)
                wait_gather(b)
                c = compute_seq(b, c)
                start_out(s, b)
            return c

        c = lax.fori_loop(0, n_main, triple_body, gb)
        for s in range(3 * n_main, seq_per_w):
            b = s % 3
            if s + 1 < seq_per_w:
                nb = (s + 1) % 3
                wait_out(nb)
                start_gather(s + 1, nb)
            wait_gather(b)
            c = compute_seq(b, c)
            start_out(s, b)
        for b in range(3):
            wait_out(b)

    return sc_kernel


def kernel(x, table, pe, gamma, beta):
    n_seq, seq_len = x.shape
    assert seq_len == SEQ_LEN
    info = plsc.get_sparse_core_info()
    nw = info.num_cores * info.num_subcores
    xc = x.astype(jnp.int32).reshape(nw, n_seq // nw, 2, HALF)
    pe2 = pe[0, :SEQ_LEN, :]
    out = _make_sc_kernel(n_seq)(xc, table, pe2, gamma, beta)
    return out.reshape(n_seq, seq_len, D_MODEL)
